# baseline (speedup 1.0000x reference)
import functools

import jax
import jax.numpy as jnp
from jax import lax
from jax.experimental import pallas as pl
from jax.experimental.pallas import tpu as pltpu
from jax.experimental.pallas import tpu_sc as plsc

HIDDEN = 64
PADDED = 128
NC = 2
NS = 16
NW = NC * NS
CHUNK = 128


def kernel(input_ids, token_embedding_weight):
    B, S = input_ids.shape
    total = B * S
    per_w = total // NW
    n_chunks = per_w // CHUNK
    idx = input_ids.reshape(NW, n_chunks, CHUNK).astype(jnp.int32)
    table128 = jnp.pad(token_embedding_weight, ((0, 0), (0, PADDED - HIDDEN)))

    mesh = plsc.VectorSubcoreMesh(core_axis_name="c", subcore_axis_name="s")

    GPC = 2
    group = GPC * CHUNK
    n_groups = per_w // group

    @functools.partial(
        pl.kernel,
        mesh=mesh,
        out_type=jax.ShapeDtypeStruct((total, PADDED), jnp.float32),
        scratch_types=[
            pltpu.VMEM((n_chunks, CHUNK), jnp.int32),
            pltpu.VMEM((2, group, PADDED), jnp.float32),
            pltpu.SemaphoreType.DMA,
            pltpu.SemaphoreType.DMA,
        ],
    )
    def emb(idx_hbm, table_hbm, out_hbm, idx_v, rows_v, gsem, osem0):
        wid = lax.axis_index("s") * NC + lax.axis_index("c")
        base = wid * per_w
        pltpu.sync_copy(idx_hbm.at[wid], idx_v)

        def fire(g, p):
            for b in range(GPC):
                pltpu.async_copy(
                    table_hbm.at[idx_v.at[g * GPC + b]],
                    rows_v.at[p, pl.ds(b * CHUNK, CHUNK)],
                    gsem,
                )

        fire(0, 0)

        def body(g, _):
            p = lax.rem(g, 2)
            for b in range(GPC):
                pltpu.make_async_copy(
                    table_hbm.at[idx_v.at[g * GPC + b]],
                    rows_v.at[p, pl.ds(b * CHUNK, CHUNK)],
                    gsem,
                ).wait()

            @pl.when(g + 1 < n_groups)
            def _():
                fire(g + 1, 1 - p)

            pltpu.async_copy(
                rows_v.at[p],
                out_hbm.at[pl.ds(base + g * group, group)],
                osem0,
            ).wait()
            return 0

        lax.fori_loop(0, n_groups, body, 0)

    out = emb(idx, table128)
    return out[:, :HIDDEN].reshape(B, S, HIDDEN)


# speedup vs baseline: 1.0072x; 1.0072x over previous
import functools

import jax
import jax.numpy as jnp
from jax import lax
from jax.experimental import pallas as pl
from jax.experimental.pallas import tpu as pltpu
from jax.experimental.pallas import tpu_sc as plsc

HIDDEN = 64
PADDED = 128
NC = 2
NS = 16
NW = NC * NS
CHUNK = 128


def kernel(input_ids, token_embedding_weight):
    B, S = input_ids.shape
    total = B * S
    per_w = total // NW
    n_chunks = per_w // CHUNK
    idx = input_ids.reshape(NW, n_chunks, CHUNK).astype(jnp.int32)
    table128 = jnp.pad(token_embedding_weight, ((0, 0), (0, PADDED - HIDDEN)))

    mesh = plsc.VectorSubcoreMesh(core_axis_name="c", subcore_axis_name="s")

    GPC = 2
    group = GPC * CHUNK
    n_groups = per_w // group

    @functools.partial(
        pl.kernel,
        mesh=mesh,
        out_type=jax.ShapeDtypeStruct((total, PADDED), jnp.float32),
        scratch_types=[
            pltpu.VMEM((n_chunks, CHUNK), jnp.int32),
            pltpu.VMEM((3, group, PADDED), jnp.float32),
            pltpu.SemaphoreType.DMA,
            pltpu.SemaphoreType.DMA,
        ],
    )
    def emb(idx_hbm, table_hbm, out_hbm, idx_v, rows_v, gsem, osem0):
        wid = lax.axis_index("s") * NC + lax.axis_index("c")
        base = wid * per_w
        pltpu.sync_copy(idx_hbm.at[wid], idx_v)

        def fire(g, p):
            for b in range(GPC):
                pltpu.async_copy(
                    table_hbm.at[idx_v.at[g * GPC + b]],
                    rows_v.at[p, pl.ds(b * CHUNK, CHUNK)],
                    gsem,
                )

        fire(0, 0)
        fire(1, 1)

        def body(g, _):
            p = lax.rem(g, 3)
            for b in range(GPC):
                pltpu.make_async_copy(
                    table_hbm.at[idx_v.at[g * GPC + b]],
                    rows_v.at[p, pl.ds(b * CHUNK, CHUNK)],
                    gsem,
                ).wait()

            @pl.when(g + 2 < n_groups)
            def _():
                fire(g + 2, lax.rem(g + 2, 3))

            pltpu.async_copy(
                rows_v.at[p],
                out_hbm.at[pl.ds(base + g * group, group)],
                osem0,
            ).wait()
            return 0

        lax.fori_loop(0, n_groups, body, 0)

    out = emb(idx, table128)
    return out[:, :HIDDEN].reshape(B, S, HIDDEN)
